# baseline (speedup 1.0000x reference)
import functools

import jax
import jax.numpy as jnp
from jax import lax
from jax.experimental import pallas as pl
from jax.experimental.pallas import tpu as pltpu
from jax.experimental.pallas import tpu_sc as plsc

_N = 10000
_E = 320000
_L = 128
_T = 3
_F = 12
_OUT = 12

_NB = 1000
_EB = 6400
_NW = 32
_PW = _E // _NW
_CH = 80
_NCH = _PW // _CH
_RPS = 624
_REM = _N - 16 * _RPS


def _wspec(shape):
    zeros = (0,) * len(shape)
    return pl.BlockSpec(shape, lambda i, _z=zeros: _z)


def _ln(z, g, beta):
    mu = jnp.mean(z, axis=-1, keepdims=True)
    d = z - mu
    var = jnp.mean(d * d, axis=-1, keepdims=True)
    return d * lax.rsqrt(var + 1e-5) * g + beta


def _dot(a, b):
    return jnp.dot(a, b, preferred_element_type=jnp.float32)


@functools.cache
def _sc_kernels():
    mesh = plsc.VectorSubcoreMesh(core_axis_name="c", subcore_axis_name="s")

    P2 = 2 * _CH
    NPAIR = _NCH // 2

    @functools.partial(
        pl.kernel,
        out_type=[jax.ShapeDtypeStruct((_E, _L), jnp.float32),
                  jax.ShapeDtypeStruct((_E, _L), jnp.float32)],
        mesh=mesh,
        scratch_types=[pltpu.VMEM((_PW,), jnp.int32),
                       pltpu.VMEM((_PW,), jnp.int32),
                       pltpu.VMEM((P2, _L), jnp.float32),
                       pltpu.VMEM((P2, _L), jnp.float32),
                       pltpu.VMEM((P2, _L), jnp.float32),
                       pltpu.VMEM((P2, _L), jnp.float32),
                       pltpu.SemaphoreType.DMA,
                       pltpu.SemaphoreType.DMA,
                       pltpu.SemaphoreType.DMA,
                       pltpu.SemaphoreType.DMA,
                       pltpu.SemaphoreType.DMA,
                       pltpu.SemaphoreType.DMA,
                       pltpu.SemaphoreType.DMA,
                       pltpu.SemaphoreType.DMA],
    )
    def gather_k(table, src, dst, out_s, out_d, idx_sv, idx_dv,
                 rows_a0, rows_b0, rows_a1, rows_b1,
                 ga0, gb0, ga1, gb1, wa0, wb0, wa1, wb1):
        wid = lax.axis_index("s") * 2 + lax.axis_index("c")
        base0 = pl.multiple_of(wid * _PW, 8)
        pltpu.sync_copy(src.at[pl.ds(base0, _PW)], idx_sv)
        pltpu.sync_copy(dst.at[pl.ds(base0, _PW)], idx_dv)

        def wb_wait(rows, out, sem):
            pltpu.make_async_copy(rows, out.at[pl.ds(0, P2)], sem).wait()

        def fire_pair(idx, loc, rows, sem):
            c0 = pltpu.async_copy(table.at[idx.at[pl.ds(loc, _CH)]],
                                  rows.at[pl.ds(0, _CH)], sem)
            c1 = pltpu.async_copy(table.at[idx.at[pl.ds(loc + _CH, _CH)]],
                                  rows.at[pl.ds(_CH, _CH)], sem)
            return c0, c1

        def body(g, carry):
            loc0 = pl.multiple_of(2 * g * P2, 8)
            loc1 = loc0 + P2
            off0 = base0 + loc0
            off1 = off0 + P2

            @pl.when(g > 0)
            def _drain0():
                wb_wait(rows_a0, out_s, wa0)
                wb_wait(rows_b0, out_d, wb0)

            a0 = fire_pair(idx_sv, loc0, rows_a0, ga0)
            b0 = fire_pair(idx_dv, loc0, rows_b0, gb0)

            @pl.when(g > 0)
            def _drain1():
                wb_wait(rows_a1, out_s, wa1)
                wb_wait(rows_b1, out_d, wb1)

            a1 = fire_pair(idx_sv, loc1, rows_a1, ga1)
            b1 = fire_pair(idx_dv, loc1, rows_b1, gb1)
            a0[0].wait()
            a0[1].wait()
            pltpu.async_copy(rows_a0, out_s.at[pl.ds(off0, P2)], wa0)
            b0[0].wait()
            b0[1].wait()
            pltpu.async_copy(rows_b0, out_d.at[pl.ds(off0, P2)], wb0)
            a1[0].wait()
            a1[1].wait()
            pltpu.async_copy(rows_a1, out_s.at[pl.ds(off1, P2)], wa1)
            b1[0].wait()
            b1[1].wait()
            pltpu.async_copy(rows_b1, out_d.at[pl.ds(off1, P2)], wb1)
            return carry

        lax.fori_loop(0, NPAIR // 2, body, 0)
        wb_wait(rows_a0, out_s, wa0)
        wb_wait(rows_b0, out_d, wb0)
        loc = pl.multiple_of((_NCH - 1) * _CH, 8)
        off = base0 + loc
        c0 = pltpu.async_copy(table.at[idx_sv.at[pl.ds(loc, _CH)]],
                              rows_a0.at[pl.ds(0, _CH)], ga0)
        c1 = pltpu.async_copy(table.at[idx_dv.at[pl.ds(loc, _CH)]],
                              rows_b0.at[pl.ds(0, _CH)], gb0)
        wb_wait(rows_a1, out_s, wa1)
        wb_wait(rows_b1, out_d, wb1)
        c0.wait()
        pltpu.sync_copy(rows_a0.at[pl.ds(0, _CH)], out_s.at[pl.ds(off, _CH)])
        c1.wait()
        pltpu.sync_copy(rows_b0.at[pl.ds(0, _CH)], out_d.at[pl.ds(off, _CH)])

    @functools.partial(
        pl.kernel,
        out_type=jax.ShapeDtypeStruct((2 * _N, _L), jnp.float32),
        mesh=mesh,
        scratch_types=[pltpu.VMEM_SHARED((_N, _L), jnp.float32),
                       pltpu.VMEM((_NCH, _CH), jnp.int32),
                       pltpu.VMEM((_CH, _L), jnp.float32),
                       pltpu.VMEM((_CH, _L), jnp.float32),
                       pltpu.SemaphoreType.DMA,
                       pltpu.SemaphoreType.DMA],
    )
    def scatter_k(msg, dstidx2, zeros_nl, out, shared, idx2_v, vals0, vals1,
                  ld0, ld1):
        cid = lax.axis_index("c")
        sid = lax.axis_index("s")
        wid = sid * 2 + cid
        row0 = pl.multiple_of(sid * _RPS, 8)
        pltpu.sync_copy(zeros_nl.at[pl.ds(row0, _RPS)],
                        shared.at[pl.ds(row0, _RPS)])

        @pl.when(sid == 0)
        def _zero_rem():
            pltpu.sync_copy(zeros_nl.at[pl.ds(16 * _RPS, _REM)],
                            shared.at[pl.ds(16 * _RPS, _REM)])

        base0 = pl.multiple_of(wid * _PW, 8)
        pltpu.sync_copy(dstidx2.at[wid], idx2_v)
        plsc.subcore_barrier()

        def ld_wait(buf, sem):
            pltpu.make_async_copy(msg.at[pl.ds(0, _CH)], buf, sem).wait()

        pltpu.async_copy(msg.at[pl.ds(base0, _CH)], vals0, ld0)

        def body(g, carry):
            k0 = 2 * g
            ld_wait(vals0, ld0)
            pltpu.async_copy(msg.at[pl.ds(base0 + (k0 + 1) * _CH, _CH)],
                             vals1, ld1)
            pltpu.sync_copy(vals0, shared.at[idx2_v.at[k0]], add=True)
            ld_wait(vals1, ld1)

            @pl.when(k0 + 2 < _NCH)
            def _next():
                pltpu.async_copy(msg.at[pl.ds(base0 + (k0 + 2) * _CH, _CH)],
                                 vals0, ld0)

            pltpu.sync_copy(vals1, shared.at[idx2_v.at[k0 + 1]], add=True)
            return carry

        lax.fori_loop(0, _NCH // 2, body, 0)
        ld_wait(vals0, ld0)
        pltpu.sync_copy(vals0, shared.at[idx2_v.at[_NCH - 1]], add=True)
        plsc.subcore_barrier()
        pltpu.sync_copy(shared.at[pl.ds(row0, _RPS)],
                        out.at[pl.ds(cid * _N + row0, _RPS)])

        @pl.when(sid == 0)
        def _out_rem():
            pltpu.sync_copy(shared.at[pl.ds(16 * _RPS, _REM)],
                            out.at[pl.ds(cid * _N + 16 * _RPS, _REM)])

    return gather_k, scatter_k


def _gather(table, src, dst):
    return _sc_kernels()[0](table, src, dst)


def _scatter(msg, dstidx, zeros_nl):
    return _sc_kernels()[1](msg, dstidx.reshape(_NW, _NCH, _CH), zeros_nl)


def _lstm_body(x_r, m_r, wih0, whh0, b0, wih1, whh1, b1, wih2, whh2, b2,
               wt1h, wt1m, bt1, wt2t, bt2, g_r, beta_r, out_r):
    xb = x_r[...]
    m = m_r[...]
    wihs = (wih0, wih1, wih2)
    whhs = (whh0, whh1, whh2)
    bs = (b0, b1, b2)
    seq = None
    h = None
    for layer in range(3):
        wihv = wihs[layer][...]
        whhv = whhs[layer][...]
        bv = bs[layer][...]
        h = jnp.zeros((_NB, _L), jnp.float32)
        c = jnp.zeros((_NB, _L), jnp.float32)
        nseq = []
        for t in range(_T):
            xt = xb[:, t * _F:(t + 1) * _F] if layer == 0 else seq[t]
            gates = _dot(xt, wihv) + _dot(h, whhv) + bv
            i = jax.nn.sigmoid(gates[:, :_L])
            f = jax.nn.sigmoid(gates[:, _L:2 * _L])
            gg = jnp.tanh(gates[:, 2 * _L:3 * _L])
            o = jax.nn.sigmoid(gates[:, 3 * _L:])
            c = f * c + i * gg
            h = o * jnp.tanh(c)
            nseq.append(h)
        seq = nseq
    z = _dot(h, wt1h[...]) + m * wt1m[...] + bt1[...]
    z = jnp.maximum(z, 0.0)
    z = _dot(z, wt2t[...]) + bt2[...]
    out_r[...] = _ln(z, g_r[...], beta_r[...])


def _lstm_tc(x36, mass2, params):
    ls = params["lstm"]
    tf = params["temp_fc"]
    wt1 = tf["lins"][0]["W"]
    args = []
    for lp in ls:
        args += [lp["W_ih"].T, lp["W_hh"].T, lp["b_ih"] + lp["b_hh"]]
    args += [wt1[:, :_L].T, wt1[:, _L], tf["lins"][0]["b"],
             tf["lins"][1]["W"].T, tf["lins"][1]["b"], tf["g"], tf["beta"]]
    in_specs = [pl.BlockSpec((_NB, _T * _F), lambda i: (i, 0)),
                pl.BlockSpec((_NB, 1), lambda i: (i, 0))]
    in_specs += [_wspec(a.shape) for a in args]
    return pl.pallas_call(
        _lstm_body,
        grid=(_N // _NB,),
        in_specs=in_specs,
        out_specs=pl.BlockSpec((_NB, _L), lambda i: (i, 0)),
        out_shape=jax.ShapeDtypeStruct((_N, _L), jnp.float32),
    )(x36, mass2, *args)


def _edge_enc_body(ea_r, w1t, b1, w2t, b2, g_r, beta_r, out_r):
    z = lax.dot_general(ea_r[...], w1t[...], (((0,), (0,)), ((), ())),
                        preferred_element_type=jnp.float32) + b1[...]
    z = jnp.maximum(z, 0.0)
    z = _dot(z, w2t[...]) + b2[...]
    out_r[...] = _ln(z, g_r[...], beta_r[...])


def _edge_enc_tc(edge_attr, p):
    args = [p["lins"][0]["W"].T, p["lins"][0]["b"],
            p["lins"][1]["W"].T, p["lins"][1]["b"], p["g"], p["beta"]]
    in_specs = [pl.BlockSpec((4, _EB), lambda i: (0, i))]
    in_specs += [_wspec(a.shape) for a in args]
    return pl.pallas_call(
        _edge_enc_body,
        grid=(_E // _EB,),
        in_specs=in_specs,
        out_specs=pl.BlockSpec((_EB, _L), lambda i: (i, 0)),
        out_shape=jax.ShapeDtypeStruct((_E, _L), jnp.float32),
    )(edge_attr.T, *args)


def _edge2_body(hd, hs, ef, wa, wb, wc, b1, w2t, b2, g_r, beta_r, msg_r, efo_r):
    hdv, hsv, efv = hd[...], hs[...], ef[...]
    wav, wbv, w2v = wa[...], wb[...], w2t[...]
    gv, bev = g_r[...], beta_r[...]
    pc = _dot(efv, wc[...]) + b1[...]
    z1 = jnp.maximum(_dot(hdv, wav) + _dot(hsv, wbv) + pc, 0.0)
    msg_r[...] = _ln(_dot(z1, w2v) + b2[...], gv, bev)
    z2 = jnp.maximum(_dot(hsv, wav) + _dot(hdv, wbv) + pc, 0.0)
    efo_r[...] = efv + _ln(_dot(z2, w2v) + b2[...], gv, bev)


def _edge1_body(hd, hs, ef, wa, wb, wc, b1, w2t, b2, g_r, beta_r, msg_r):
    pc = _dot(ef[...], wc[...]) + b1[...]
    z1 = jnp.maximum(_dot(hd[...], wa[...]) + _dot(hs[...], wb[...]) + pc, 0.0)
    msg_r[...] = _ln(_dot(z1, w2t[...]) + b2[...], g_r[...], beta_r[...])


def _edge_rad_body(hd, hs, wa, wb, b1, w2t, b2, g_r, beta_r, msg_r):
    z1 = jnp.maximum(_dot(hd[...], wa[...]) + _dot(hs[...], wb[...]) + b1[...],
                     0.0)
    msg_r[...] = _ln(_dot(z1, w2t[...]) + b2[...], g_r[...], beta_r[...])


def _edge_weight_args(p):
    w1 = p["lins"][0]["W"]
    return [w1[:, :_L].T, w1[:, _L:2 * _L].T, w1[:, 2 * _L:].T,
            p["lins"][0]["b"], p["lins"][1]["W"].T, p["lins"][1]["b"],
            p["g"], p["beta"]]


def _edge_mlp_tc(hd_g, hs_g, ef, p, want_ef):
    args = _edge_weight_args(p)
    in_specs = [pl.BlockSpec((_EB, _L), lambda i: (i, 0)) for _ in range(3)]
    in_specs += [_wspec(a.shape) for a in args]
    eb_spec = pl.BlockSpec((_EB, _L), lambda i: (i, 0))
    eb_shape = jax.ShapeDtypeStruct((_E, _L), jnp.float32)
    if want_ef:
        return pl.pallas_call(
            _edge2_body,
            grid=(_E // _EB,),
            in_specs=in_specs,
            out_specs=[eb_spec, eb_spec],
            out_shape=[eb_shape, eb_shape],
        )(hd_g, hs_g, ef, *args)
    return pl.pallas_call(
        _edge1_body,
        grid=(_E // _EB,),
        in_specs=in_specs,
        out_specs=eb_spec,
        out_shape=eb_shape,
    )(hd_g, hs_g, ef, *args)


def _edge_rad_tc(hd_g, hs_g, p):
    w1 = p["lins"][0]["W"]
    args = [w1[:, :_L].T, w1[:, _L:2 * _L].T, p["lins"][0]["b"],
            p["lins"][1]["W"].T, p["lins"][1]["b"], p["g"], p["beta"]]
    in_specs = [pl.BlockSpec((_EB, _L), lambda i: (i, 0)) for _ in range(2)]
    in_specs += [_wspec(a.shape) for a in args]
    return pl.pallas_call(
        _edge_rad_body,
        grid=(_E // _EB,),
        in_specs=in_specs,
        out_specs=pl.BlockSpec((_EB, _L), lambda i: (i, 0)),
        out_shape=jax.ShapeDtypeStruct((_E, _L), jnp.float32),
    )(hd_g, hs_g, *args)


def _node_body(a0, a1, h_r, wa, wb, b1, w2t, b2, g_r, beta_r, out_r):
    hv = h_r[...]
    aggr = a0[...] + a1[...]
    z = jnp.maximum(_dot(aggr, wa[...]) + _dot(hv, wb[...]) + b1[...], 0.0)
    z = _dot(z, w2t[...]) + b2[...]
    out_r[...] = hv + _ln(z, g_r[...], beta_r[...])


def _node_tc(parts, h, p):
    w1 = p["lins"][0]["W"]
    args = [w1[:, :_L].T, w1[:, _L:].T, p["lins"][0]["b"],
            p["lins"][1]["W"].T, p["lins"][1]["b"], p["g"], p["beta"]]
    nblk = _N // _NB
    in_specs = [pl.BlockSpec((_NB, _L), lambda i: (i, 0)),
                pl.BlockSpec((_NB, _L), lambda i, _o=nblk: (i + _o, 0)),
                pl.BlockSpec((_NB, _L), lambda i: (i, 0))]
    in_specs += [_wspec(a.shape) for a in args]
    return pl.pallas_call(
        _node_body,
        grid=(nblk,),
        in_specs=in_specs,
        out_specs=pl.BlockSpec((_NB, _L), lambda i: (i, 0)),
        out_shape=jax.ShapeDtypeStruct((_N, _L), jnp.float32),
    )(parts, parts, h, *args)


def _final_body(ht, hr, wapa, wapb, bap, gap, betap, wd1t, bd1, wd2t, bd2,
                gd, betad, out_r):
    z = _dot(ht[...], wapa[...]) + _dot(hr[...], wapb[...]) + bap[...]
    hcat = _ln(z, gap[...], betap[...])
    z2 = jnp.maximum(_dot(hcat, wd1t[...]) + bd1[...], 0.0)
    z2 = _dot(z2, wd2t[...]) + bd2[...]
    out_r[...] = _ln(z2, gd[...], betad[...])


def _final_tc(h_topo, h_rad, pap, pdec):
    wap = pap["lins"][0]["W"]
    args = [wap[:, :_L].T, wap[:, _L:].T, pap["lins"][0]["b"],
            pap["g"], pap["beta"],
            pdec["lins"][0]["W"].T, pdec["lins"][0]["b"],
            pdec["lins"][1]["W"].T, pdec["lins"][1]["b"],
            pdec["g"], pdec["beta"]]
    in_specs = [pl.BlockSpec((_NB, _L), lambda i: (i, 0)),
                pl.BlockSpec((_NB, _L), lambda i: (i, 0))]
    in_specs += [_wspec(a.shape) for a in args]
    return pl.pallas_call(
        _final_body,
        grid=(_N // _NB,),
        in_specs=in_specs,
        out_specs=pl.BlockSpec((_NB, _OUT), lambda i: (i, 0)),
        out_shape=jax.ShapeDtypeStruct((_N, _OUT), jnp.float32),
    )(h_topo, h_rad, *args)


def kernel(x, node_mass, edge_attr, params, edge_index, radius_edges):
    x36 = jnp.transpose(x, (0, 2, 1)).reshape(_N, _T * _F)
    mass2 = node_mass[:, None]
    src = edge_index[0].astype(jnp.int32)
    dst = edge_index[1].astype(jnp.int32)
    rsrc = radius_edges[0].astype(jnp.int32)
    rdst = radius_edges[1].astype(jnp.int32)

    h0 = _lstm_tc(x36, mass2, params)
    ef = _edge_enc_tc(edge_attr, params["edge_enc"])
    zeros_nl = jnp.zeros((_N, _L), jnp.float32)

    pe, pn = params["topo_edge"], params["topo_node"]
    h = h0
    for i in range(3):
        hs_g, hd_g = _gather(h, src, dst)
        if i < 2:
            msg, ef = _edge_mlp_tc(hd_g, hs_g, ef, pe, True)
        else:
            msg = _edge_mlp_tc(hd_g, hs_g, ef, pe, False)
        parts = _scatter(msg, dst, zeros_nl)
        h = _node_tc(parts, h, pn)

    hs_r, hd_r = _gather(h0, rsrc, rdst)
    msg_r = _edge_rad_tc(hd_r, hs_r, params["rad_edge"])
    parts_r = _scatter(msg_r, rdst, zeros_nl)
    h_rad = _node_tc(parts_r, h0, params["rad_node"])

    return _final_tc(h, h_rad, params["add_passage"], params["node_dec"])


# speedup vs baseline: 1.0021x; 1.0021x over previous
import functools

import jax
import jax.numpy as jnp
from jax import lax
from jax.experimental import pallas as pl
from jax.experimental.pallas import tpu as pltpu
from jax.experimental.pallas import tpu_sc as plsc

_N = 10000
_E = 320000
_L = 128
_T = 3
_F = 12
_OUT = 12

_NB = 1000
_EB = 6400
_NW = 32
_PW = _E // _NW
_CH = 80
_NCH = _PW // _CH
_RPS = 624
_REM = _N - 16 * _RPS


def _wspec(shape):
    zeros = (0,) * len(shape)
    return pl.BlockSpec(shape, lambda i, _z=zeros: _z)


def _ln(z, g, beta):
    mu = jnp.mean(z, axis=-1, keepdims=True)
    d = z - mu
    var = jnp.mean(d * d, axis=-1, keepdims=True)
    return d * lax.rsqrt(var + 1e-5) * g + beta


def _dot(a, b):
    return jnp.dot(a, b, preferred_element_type=jnp.float32)


@functools.cache
def _sc_kernels():
    mesh = plsc.VectorSubcoreMesh(core_axis_name="c", subcore_axis_name="s")

    P2 = 2 * _CH
    NPAIR = _NCH // 2

    @functools.partial(
        pl.kernel,
        out_type=[jax.ShapeDtypeStruct((_E, _L), jnp.float32),
                  jax.ShapeDtypeStruct((_E, _L), jnp.float32)],
        mesh=mesh,
        scratch_types=[pltpu.VMEM((_PW,), jnp.int32),
                       pltpu.VMEM((_PW,), jnp.int32),
                       pltpu.VMEM((P2, _L), jnp.float32),
                       pltpu.VMEM((P2, _L), jnp.float32),
                       pltpu.VMEM((P2, _L), jnp.float32),
                       pltpu.VMEM((P2, _L), jnp.float32),
                       pltpu.SemaphoreType.DMA,
                       pltpu.SemaphoreType.DMA,
                       pltpu.SemaphoreType.DMA,
                       pltpu.SemaphoreType.DMA,
                       pltpu.SemaphoreType.DMA,
                       pltpu.SemaphoreType.DMA,
                       pltpu.SemaphoreType.DMA,
                       pltpu.SemaphoreType.DMA],
    )
    def gather_k(table, src, dst, out_s, out_d, idx_sv, idx_dv,
                 rows_a0, rows_b0, rows_a1, rows_b1,
                 ga0, gb0, ga1, gb1, wa0, wb0, wa1, wb1):
        wid = lax.axis_index("s") * 2 + lax.axis_index("c")
        base0 = pl.multiple_of(wid * _PW, 8)
        pltpu.sync_copy(src.at[pl.ds(base0, _PW)], idx_sv)
        pltpu.sync_copy(dst.at[pl.ds(base0, _PW)], idx_dv)

        def wb_wait(rows, out, sem):
            pltpu.make_async_copy(rows, out.at[pl.ds(0, P2)], sem).wait()

        def fire_pair(idx, loc, rows, sem):
            c0 = pltpu.async_copy(table.at[idx.at[pl.ds(loc, _CH)]],
                                  rows.at[pl.ds(0, _CH)], sem)
            c1 = pltpu.async_copy(table.at[idx.at[pl.ds(loc + _CH, _CH)]],
                                  rows.at[pl.ds(_CH, _CH)], sem)
            return c0, c1

        def body(g, carry):
            loc0 = pl.multiple_of(2 * g * P2, 8)
            loc1 = loc0 + P2
            off0 = base0 + loc0
            off1 = off0 + P2

            @pl.when(g > 0)
            def _drain0():
                wb_wait(rows_a0, out_s, wa0)
                wb_wait(rows_b0, out_d, wb0)

            a0 = fire_pair(idx_sv, loc0, rows_a0, ga0)
            b0 = fire_pair(idx_dv, loc0, rows_b0, gb0)

            @pl.when(g > 0)
            def _drain1():
                wb_wait(rows_a1, out_s, wa1)
                wb_wait(rows_b1, out_d, wb1)

            a1 = fire_pair(idx_sv, loc1, rows_a1, ga1)
            b1 = fire_pair(idx_dv, loc1, rows_b1, gb1)
            a0[0].wait()
            a0[1].wait()
            pltpu.async_copy(rows_a0, out_s.at[pl.ds(off0, P2)], wa0)
            b0[0].wait()
            b0[1].wait()
            pltpu.async_copy(rows_b0, out_d.at[pl.ds(off0, P2)], wb0)
            a1[0].wait()
            a1[1].wait()
            pltpu.async_copy(rows_a1, out_s.at[pl.ds(off1, P2)], wa1)
            b1[0].wait()
            b1[1].wait()
            pltpu.async_copy(rows_b1, out_d.at[pl.ds(off1, P2)], wb1)
            return carry

        lax.fori_loop(0, NPAIR // 2, body, 0)
        wb_wait(rows_a0, out_s, wa0)
        wb_wait(rows_b0, out_d, wb0)
        loc = pl.multiple_of((_NCH - 1) * _CH, 8)
        off = base0 + loc
        c0 = pltpu.async_copy(table.at[idx_sv.at[pl.ds(loc, _CH)]],
                              rows_a0.at[pl.ds(0, _CH)], ga0)
        c1 = pltpu.async_copy(table.at[idx_dv.at[pl.ds(loc, _CH)]],
                              rows_b0.at[pl.ds(0, _CH)], gb0)
        wb_wait(rows_a1, out_s, wa1)
        wb_wait(rows_b1, out_d, wb1)
        c0.wait()
        pltpu.sync_copy(rows_a0.at[pl.ds(0, _CH)], out_s.at[pl.ds(off, _CH)])
        c1.wait()
        pltpu.sync_copy(rows_b0.at[pl.ds(0, _CH)], out_d.at[pl.ds(off, _CH)])

    @functools.partial(
        pl.kernel,
        out_type=jax.ShapeDtypeStruct((2 * _N, _L), jnp.float32),
        mesh=mesh,
        scratch_types=[pltpu.VMEM_SHARED((_N, _L), jnp.float32),
                       pltpu.VMEM((_NCH, _CH), jnp.int32),
                       pltpu.VMEM((_CH, _L), jnp.float32),
                       pltpu.VMEM((_CH, _L), jnp.float32),
                       pltpu.SemaphoreType.DMA,
                       pltpu.SemaphoreType.DMA],
    )
    def scatter_k(msg, dstidx2, zeros_nl, out, shared, idx2_v, vals0, vals1,
                  ld0, ld1):
        cid = lax.axis_index("c")
        sid = lax.axis_index("s")
        wid = sid * 2 + cid
        row0 = pl.multiple_of(sid * _RPS, 8)
        pltpu.sync_copy(zeros_nl.at[pl.ds(row0, _RPS)],
                        shared.at[pl.ds(row0, _RPS)])

        @pl.when(sid == 0)
        def _zero_rem():
            pltpu.sync_copy(zeros_nl.at[pl.ds(16 * _RPS, _REM)],
                            shared.at[pl.ds(16 * _RPS, _REM)])

        base0 = pl.multiple_of(wid * _PW, 8)
        pltpu.sync_copy(dstidx2.at[wid], idx2_v)
        plsc.subcore_barrier()

        def ld_wait(buf, sem):
            pltpu.make_async_copy(msg.at[pl.ds(0, _CH)], buf, sem).wait()

        pltpu.async_copy(msg.at[pl.ds(base0, _CH)], vals0, ld0)

        def body(g, carry):
            k0 = 2 * g
            ld_wait(vals0, ld0)
            pltpu.async_copy(msg.at[pl.ds(base0 + (k0 + 1) * _CH, _CH)],
                             vals1, ld1)
            pltpu.sync_copy(vals0, shared.at[idx2_v.at[k0]], add=True)
            ld_wait(vals1, ld1)

            @pl.when(k0 + 2 < _NCH)
            def _next():
                pltpu.async_copy(msg.at[pl.ds(base0 + (k0 + 2) * _CH, _CH)],
                                 vals0, ld0)

            pltpu.sync_copy(vals1, shared.at[idx2_v.at[k0 + 1]], add=True)
            return carry

        lax.fori_loop(0, _NCH // 2, body, 0)
        ld_wait(vals0, ld0)
        pltpu.sync_copy(vals0, shared.at[idx2_v.at[_NCH - 1]], add=True)
        plsc.subcore_barrier()
        pltpu.sync_copy(shared.at[pl.ds(row0, _RPS)],
                        out.at[pl.ds(cid * _N + row0, _RPS)])

        @pl.when(sid == 0)
        def _out_rem():
            pltpu.sync_copy(shared.at[pl.ds(16 * _RPS, _REM)],
                            out.at[pl.ds(cid * _N + 16 * _RPS, _REM)])

    return gather_k, scatter_k


def _gather(table, src, dst):
    return _sc_kernels()[0](table, src, dst)


def _scatter(msg, dstidx, zeros_nl):
    return _sc_kernels()[1](msg, dstidx.reshape(_NW, _NCH, _CH), zeros_nl)


def _lstm_body(x_r, m_r, wih0, whh0, b0, wih1, whh1, b1, wih2, whh2, b2,
               wt1h, wt1m, bt1, wt2t, bt2, g_r, beta_r, out_r):
    xb = x_r[...]
    m = m_r[...]
    wihs = (wih0, wih1, wih2)
    whhs = (whh0, whh1, whh2)
    bs = (b0, b1, b2)
    seq = None
    h = None
    for layer in range(3):
        wihv = wihs[layer][...]
        whhv = whhs[layer][...]
        bv = bs[layer][...]
        h = jnp.zeros((_NB, _L), jnp.float32)
        c = jnp.zeros((_NB, _L), jnp.float32)
        nseq = []
        for t in range(_T):
            xt = xb[:, t * _F:(t + 1) * _F] if layer == 0 else seq[t]
            gates = _dot(xt, wihv) + _dot(h, whhv) + bv
            i = jax.nn.sigmoid(gates[:, :_L])
            f = jax.nn.sigmoid(gates[:, _L:2 * _L])
            gg = jnp.tanh(gates[:, 2 * _L:3 * _L])
            o = jax.nn.sigmoid(gates[:, 3 * _L:])
            c = f * c + i * gg
            h = o * jnp.tanh(c)
            nseq.append(h)
        seq = nseq
    z = _dot(h, wt1h[...]) + m * wt1m[...] + bt1[...]
    z = jnp.maximum(z, 0.0)
    z = _dot(z, wt2t[...]) + bt2[...]
    out_r[...] = _ln(z, g_r[...], beta_r[...])


def _lstm_tc(x36, mass2, params):
    ls = params["lstm"]
    tf = params["temp_fc"]
    wt1 = tf["lins"][0]["W"]
    args = []
    for lp in ls:
        args += [lp["W_ih"].T, lp["W_hh"].T, lp["b_ih"] + lp["b_hh"]]
    args += [wt1[:, :_L].T, wt1[:, _L], tf["lins"][0]["b"],
             tf["lins"][1]["W"].T, tf["lins"][1]["b"], tf["g"], tf["beta"]]
    in_specs = [pl.BlockSpec((_NB, _T * _F), lambda i: (i, 0)),
                pl.BlockSpec((_NB, 1), lambda i: (i, 0))]
    in_specs += [_wspec(a.shape) for a in args]
    return pl.pallas_call(
        _lstm_body,
        grid=(_N // _NB,),
        in_specs=in_specs,
        out_specs=pl.BlockSpec((_NB, _L), lambda i: (i, 0)),
        out_shape=jax.ShapeDtypeStruct((_N, _L), jnp.float32),
    )(x36, mass2, *args)


def _edge_enc_body(ea_r, w1t, b1, w2t, b2, g_r, beta_r, out_r):
    z = lax.dot_general(ea_r[...], w1t[...], (((0,), (0,)), ((), ())),
                        preferred_element_type=jnp.float32) + b1[...]
    z = jnp.maximum(z, 0.0)
    z = _dot(z, w2t[...]) + b2[...]
    out_r[...] = _ln(z, g_r[...], beta_r[...])


def _edge_enc_tc(edge_attr, p):
    args = [p["lins"][0]["W"].T, p["lins"][0]["b"],
            p["lins"][1]["W"].T, p["lins"][1]["b"], p["g"], p["beta"]]
    in_specs = [pl.BlockSpec((4, _EB), lambda i: (0, i))]
    in_specs += [_wspec(a.shape) for a in args]
    return pl.pallas_call(
        _edge_enc_body,
        grid=(_E // _EB,),
        in_specs=in_specs,
        out_specs=pl.BlockSpec((_EB, _L), lambda i: (i, 0)),
        out_shape=jax.ShapeDtypeStruct((_E, _L), jnp.float32),
    )(edge_attr.T, *args)


def _edge2_body(hd, hs, ef, wa, wb, wc, b1, w2t, b2, g_r, beta_r, msg_r, efo_r):
    hdv, hsv, efv = hd[...], hs[...], ef[...]
    wav, wbv, w2v = wa[...], wb[...], w2t[...]
    gv, bev = g_r[...], beta_r[...]
    pc = _dot(efv, wc[...]) + b1[...]
    z1 = jnp.maximum(_dot(hdv, wav) + _dot(hsv, wbv) + pc, 0.0)
    msg_r[...] = _ln(_dot(z1, w2v) + b2[...], gv, bev)
    z2 = jnp.maximum(_dot(hsv, wav) + _dot(hdv, wbv) + pc, 0.0)
    efo_r[...] = efv + _ln(_dot(z2, w2v) + b2[...], gv, bev)


def _edge1_body(hd, hs, ef, wa, wb, wc, b1, w2t, b2, g_r, beta_r, msg_r):
    pc = _dot(ef[...], wc[...]) + b1[...]
    z1 = jnp.maximum(_dot(hd[...], wa[...]) + _dot(hs[...], wb[...]) + pc, 0.0)
    msg_r[...] = _ln(_dot(z1, w2t[...]) + b2[...], g_r[...], beta_r[...])


def _edge_rad_body(hd, hs, wa, wb, b1, w2t, b2, g_r, beta_r, msg_r):
    z1 = jnp.maximum(_dot(hd[...], wa[...]) + _dot(hs[...], wb[...]) + b1[...],
                     0.0)
    msg_r[...] = _ln(_dot(z1, w2t[...]) + b2[...], g_r[...], beta_r[...])


def _edge_weight_args(p):
    w1 = p["lins"][0]["W"]
    return [w1[:, :_L].T, w1[:, _L:2 * _L].T, w1[:, 2 * _L:].T,
            p["lins"][0]["b"], p["lins"][1]["W"].T, p["lins"][1]["b"],
            p["g"], p["beta"]]


def _edge_mlp_tc(hd_g, hs_g, ef, p, want_ef):
    args = _edge_weight_args(p)
    in_specs = [pl.BlockSpec((_EB, _L), lambda i: (i, 0)) for _ in range(3)]
    in_specs += [_wspec(a.shape) for a in args]
    eb_spec = pl.BlockSpec((_EB, _L), lambda i: (i, 0))
    eb_shape = jax.ShapeDtypeStruct((_E, _L), jnp.float32)
    if want_ef:
        return pl.pallas_call(
            _edge2_body,
            grid=(_E // _EB,),
            in_specs=in_specs,
            out_specs=[eb_spec, eb_spec],
            out_shape=[eb_shape, eb_shape],
        )(hd_g, hs_g, ef, *args)
    return pl.pallas_call(
        _edge1_body,
        grid=(_E // _EB,),
        in_specs=in_specs,
        out_specs=eb_spec,
        out_shape=eb_shape,
    )(hd_g, hs_g, ef, *args)


def _edge_rad_tc(hd_g, hs_g, p):
    w1 = p["lins"][0]["W"]
    args = [w1[:, :_L].T, w1[:, _L:2 * _L].T, p["lins"][0]["b"],
            p["lins"][1]["W"].T, p["lins"][1]["b"], p["g"], p["beta"]]
    in_specs = [pl.BlockSpec((_EB, _L), lambda i: (i, 0)) for _ in range(2)]
    in_specs += [_wspec(a.shape) for a in args]
    return pl.pallas_call(
        _edge_rad_body,
        grid=(_E // _EB,),
        in_specs=in_specs,
        out_specs=pl.BlockSpec((_EB, _L), lambda i: (i, 0)),
        out_shape=jax.ShapeDtypeStruct((_E, _L), jnp.float32),
    )(hd_g, hs_g, *args)


def _node_body(a0, a1, h_r, wa, wb, b1, w2t, b2, g_r, beta_r, out_r):
    hv = h_r[...]
    aggr = a0[...] + a1[...]
    z = jnp.maximum(_dot(aggr, wa[...]) + _dot(hv, wb[...]) + b1[...], 0.0)
    z = _dot(z, w2t[...]) + b2[...]
    out_r[...] = hv + _ln(z, g_r[...], beta_r[...])


def _node_tc(parts, h, p):
    w1 = p["lins"][0]["W"]
    args = [w1[:, :_L].T, w1[:, _L:].T, p["lins"][0]["b"],
            p["lins"][1]["W"].T, p["lins"][1]["b"], p["g"], p["beta"]]
    nblk = _N // _NB
    in_specs = [pl.BlockSpec((_NB, _L), lambda i: (i, 0)),
                pl.BlockSpec((_NB, _L), lambda i, _o=nblk: (i + _o, 0)),
                pl.BlockSpec((_NB, _L), lambda i: (i, 0))]
    in_specs += [_wspec(a.shape) for a in args]
    return pl.pallas_call(
        _node_body,
        grid=(nblk,),
        in_specs=in_specs,
        out_specs=pl.BlockSpec((_NB, _L), lambda i: (i, 0)),
        out_shape=jax.ShapeDtypeStruct((_N, _L), jnp.float32),
    )(parts, parts, h, *args)


def _final_body(ht, hr, wapa, wapb, bap, gap, betap, wd1t, bd1, wd2t, bd2,
                gd, betad, out_r):
    z = _dot(ht[...], wapa[...]) + _dot(hr[...], wapb[...]) + bap[...]
    hcat = _ln(z, gap[...], betap[...])
    z2 = jnp.maximum(_dot(hcat, wd1t[...]) + bd1[...], 0.0)
    z2 = _dot(z2, wd2t[...]) + bd2[...]
    out_r[...] = _ln(z2, gd[...], betad[...])


def _final_tc(h_topo, h_rad, pap, pdec):
    wap = pap["lins"][0]["W"]
    args = [wap[:, :_L].T, wap[:, _L:].T, pap["lins"][0]["b"],
            pap["g"], pap["beta"],
            pdec["lins"][0]["W"].T, pdec["lins"][0]["b"],
            pdec["lins"][1]["W"].T, pdec["lins"][1]["b"],
            pdec["g"], pdec["beta"]]
    in_specs = [pl.BlockSpec((_NB, _L), lambda i: (i, 0)),
                pl.BlockSpec((_NB, _L), lambda i: (i, 0))]
    in_specs += [_wspec(a.shape) for a in args]
    return pl.pallas_call(
        _final_body,
        grid=(_N // _NB,),
        in_specs=in_specs,
        out_specs=pl.BlockSpec((_NB, _OUT), lambda i: (i, 0)),
        out_shape=jax.ShapeDtypeStruct((_N, _OUT), jnp.float32),
    )(h_topo, h_rad, *args)


def kernel(x, node_mass, edge_attr, params, edge_index, radius_edges):
    x36 = jnp.transpose(x, (0, 2, 1)).reshape(_N, _T * _F)
    mass2 = node_mass[:, None]
    src = edge_index[0].astype(jnp.int32)
    dst = edge_index[1].astype(jnp.int32)
    rsrc = radius_edges[0].astype(jnp.int32)
    rdst = radius_edges[1].astype(jnp.int32)

    h0 = _lstm_tc(x36, mass2, params)
    ef = _edge_enc_tc(edge_attr, params["edge_enc"])
    zeros_nl = jnp.zeros((_N, _L), jnp.float32)

    pe, pn = params["topo_edge"], params["topo_node"]
    h = h0

    hs_g, hd_g = _gather(h, src, dst)
    hs_r, hd_r = _gather(h0, rsrc, rdst)
    msg, ef = _edge_mlp_tc(hd_g, hs_g, ef, pe, True)
    msg_r = _edge_rad_tc(hd_r, hs_r, params["rad_edge"])
    parts = _scatter(msg, dst, zeros_nl)
    h = _node_tc(parts, h, pn)

    hs_g, hd_g = _gather(h, src, dst)
    parts_r = _scatter(msg_r, rdst, zeros_nl)
    msg, ef = _edge_mlp_tc(hd_g, hs_g, ef, pe, True)
    parts = _scatter(msg, dst, zeros_nl)
    h = _node_tc(parts, h, pn)
    h_rad = _node_tc(parts_r, h0, params["rad_node"])

    hs_g, hd_g = _gather(h, src, dst)
    msg = _edge_mlp_tc(hd_g, hs_g, ef, pe, False)
    parts = _scatter(msg, dst, zeros_nl)
    h = _node_tc(parts, h, pn)

    return _final_tc(h, h_rad, params["add_passage"], params["node_dec"])


# speedup vs baseline: 1.0130x; 1.0108x over previous
import functools

import jax
import jax.numpy as jnp
from jax import lax
from jax.experimental import pallas as pl
from jax.experimental.pallas import tpu as pltpu
from jax.experimental.pallas import tpu_sc as plsc

_N = 10000
_E = 320000
_L = 128
_T = 3
_F = 12
_OUT = 12

_NB = 1000
_EB = 6400
_NW = 32
_PW = _E // _NW
_CH = 80
_NCH = _PW // _CH
_RPS = 624
_REM = _N - 16 * _RPS


def _wspec(shape):
    zeros = (0,) * len(shape)
    return pl.BlockSpec(shape, lambda i, _z=zeros: _z)


def _ln(z, g, beta):
    mu = jnp.mean(z, axis=-1, keepdims=True)
    d = z - mu
    var = jnp.mean(d * d, axis=-1, keepdims=True)
    return d * lax.rsqrt(var + 1e-5) * g + beta


def _dot(a, b):
    return jnp.dot(a, b, preferred_element_type=jnp.float32)


@functools.cache
def _sc_kernels():
    mesh = plsc.VectorSubcoreMesh(core_axis_name="c", subcore_axis_name="s")

    P2 = 2 * _CH
    NPAIR = _NCH // 2

    @functools.partial(
        pl.kernel,
        out_type=[jax.ShapeDtypeStruct((_E, _L), jnp.float32),
                  jax.ShapeDtypeStruct((_E, _L), jnp.float32)],
        mesh=mesh,
        scratch_types=[pltpu.VMEM((_PW,), jnp.int32),
                       pltpu.VMEM((_PW,), jnp.int32),
                       pltpu.VMEM((P2, _L), jnp.float32),
                       pltpu.VMEM((P2, _L), jnp.float32),
                       pltpu.VMEM((P2, _L), jnp.float32),
                       pltpu.VMEM((P2, _L), jnp.float32),
                       pltpu.SemaphoreType.DMA,
                       pltpu.SemaphoreType.DMA,
                       pltpu.SemaphoreType.DMA,
                       pltpu.SemaphoreType.DMA,
                       pltpu.SemaphoreType.DMA,
                       pltpu.SemaphoreType.DMA,
                       pltpu.SemaphoreType.DMA,
                       pltpu.SemaphoreType.DMA],
    )
    def gather_k(table, src, dst, out_s, out_d, idx_sv, idx_dv,
                 rows_a0, rows_b0, rows_a1, rows_b1,
                 ga0, gb0, ga1, gb1, wa0, wb0, wa1, wb1):
        wid = lax.axis_index("s") * 2 + lax.axis_index("c")
        base0 = pl.multiple_of(wid * _PW, 8)
        pltpu.sync_copy(src.at[pl.ds(base0, _PW)], idx_sv)
        pltpu.sync_copy(dst.at[pl.ds(base0, _PW)], idx_dv)

        def wb_wait(rows, out, sem):
            pltpu.make_async_copy(rows, out.at[pl.ds(0, P2)], sem).wait()

        def fire_pair(idx, loc, rows, sem):
            c0 = pltpu.async_copy(table.at[idx.at[pl.ds(loc, _CH)]],
                                  rows.at[pl.ds(0, _CH)], sem)
            c1 = pltpu.async_copy(table.at[idx.at[pl.ds(loc + _CH, _CH)]],
                                  rows.at[pl.ds(_CH, _CH)], sem)
            return c0, c1

        def body(g, carry):
            loc0 = pl.multiple_of(2 * g * P2, 8)
            loc1 = loc0 + P2
            off0 = base0 + loc0
            off1 = off0 + P2

            @pl.when(g > 0)
            def _drain0():
                wb_wait(rows_a0, out_s, wa0)
                wb_wait(rows_b0, out_d, wb0)

            a0 = fire_pair(idx_sv, loc0, rows_a0, ga0)
            b0 = fire_pair(idx_dv, loc0, rows_b0, gb0)

            @pl.when(g > 0)
            def _drain1():
                wb_wait(rows_a1, out_s, wa1)
                wb_wait(rows_b1, out_d, wb1)

            a1 = fire_pair(idx_sv, loc1, rows_a1, ga1)
            b1 = fire_pair(idx_dv, loc1, rows_b1, gb1)
            a0[0].wait()
            a0[1].wait()
            pltpu.async_copy(rows_a0, out_s.at[pl.ds(off0, P2)], wa0)
            b0[0].wait()
            b0[1].wait()
            pltpu.async_copy(rows_b0, out_d.at[pl.ds(off0, P2)], wb0)
            a1[0].wait()
            a1[1].wait()
            pltpu.async_copy(rows_a1, out_s.at[pl.ds(off1, P2)], wa1)
            b1[0].wait()
            b1[1].wait()
            pltpu.async_copy(rows_b1, out_d.at[pl.ds(off1, P2)], wb1)
            return carry

        lax.fori_loop(0, NPAIR // 2, body, 0)
        wb_wait(rows_a0, out_s, wa0)
        wb_wait(rows_b0, out_d, wb0)
        loc = pl.multiple_of((_NCH - 1) * _CH, 8)
        off = base0 + loc
        c0 = pltpu.async_copy(table.at[idx_sv.at[pl.ds(loc, _CH)]],
                              rows_a0.at[pl.ds(0, _CH)], ga0)
        c1 = pltpu.async_copy(table.at[idx_dv.at[pl.ds(loc, _CH)]],
                              rows_b0.at[pl.ds(0, _CH)], gb0)
        wb_wait(rows_a1, out_s, wa1)
        wb_wait(rows_b1, out_d, wb1)
        c0.wait()
        pltpu.sync_copy(rows_a0.at[pl.ds(0, _CH)], out_s.at[pl.ds(off, _CH)])
        c1.wait()
        pltpu.sync_copy(rows_b0.at[pl.ds(0, _CH)], out_d.at[pl.ds(off, _CH)])

    @functools.partial(
        pl.kernel,
        out_type=jax.ShapeDtypeStruct((2 * _N, _L), jnp.float32),
        mesh=mesh,
        scratch_types=[pltpu.VMEM_SHARED((_N, _L), jnp.float32),
                       pltpu.VMEM((_NCH, _CH), jnp.int32),
                       pltpu.VMEM((_CH, _L), jnp.float32),
                       pltpu.VMEM((_CH, _L), jnp.float32),
                       pltpu.SemaphoreType.DMA,
                       pltpu.SemaphoreType.DMA],
    )
    def scatter_k(msg, dstidx2, zeros_nl, out, shared, idx2_v, vals0, vals1,
                  ld0, ld1):
        cid = lax.axis_index("c")
        sid = lax.axis_index("s")
        wid = sid * 2 + cid
        row0 = pl.multiple_of(sid * _RPS, 8)
        pltpu.sync_copy(zeros_nl.at[pl.ds(row0, _RPS)],
                        shared.at[pl.ds(row0, _RPS)])

        @pl.when(sid == 0)
        def _zero_rem():
            pltpu.sync_copy(zeros_nl.at[pl.ds(16 * _RPS, _REM)],
                            shared.at[pl.ds(16 * _RPS, _REM)])

        base0 = pl.multiple_of(wid * _PW, 8)
        pltpu.sync_copy(dstidx2.at[wid], idx2_v)
        plsc.subcore_barrier()

        def ld_wait(buf, sem):
            pltpu.make_async_copy(msg.at[pl.ds(0, _CH)], buf, sem).wait()

        pltpu.async_copy(msg.at[pl.ds(base0, _CH)], vals0, ld0)

        def body(g, carry):
            k0 = 2 * g
            ld_wait(vals0, ld0)
            pltpu.async_copy(msg.at[pl.ds(base0 + (k0 + 1) * _CH, _CH)],
                             vals1, ld1)
            pltpu.sync_copy(vals0, shared.at[idx2_v.at[k0]], add=True)
            ld_wait(vals1, ld1)

            @pl.when(k0 + 2 < _NCH)
            def _next():
                pltpu.async_copy(msg.at[pl.ds(base0 + (k0 + 2) * _CH, _CH)],
                                 vals0, ld0)

            pltpu.sync_copy(vals1, shared.at[idx2_v.at[k0 + 1]], add=True)
            return carry

        lax.fori_loop(0, _NCH // 2, body, 0)
        ld_wait(vals0, ld0)
        pltpu.sync_copy(vals0, shared.at[idx2_v.at[_NCH - 1]], add=True)
        plsc.subcore_barrier()
        pltpu.sync_copy(shared.at[pl.ds(row0, _RPS)],
                        out.at[pl.ds(cid * _N + row0, _RPS)])

        @pl.when(sid == 0)
        def _out_rem():
            pltpu.sync_copy(shared.at[pl.ds(16 * _RPS, _REM)],
                            out.at[pl.ds(cid * _N + 16 * _RPS, _REM)])

    return gather_k, scatter_k


def _gather(table, src, dst):
    return _sc_kernels()[0](table, src, dst)


def _scatter(msg, dstidx, zeros_nl):
    return _sc_kernels()[1](msg, dstidx.reshape(_NW, _NCH, _CH), zeros_nl)


def _lstm_body(x_r, m_r, wih0, whh0, b0, wih1, whh1, b1, wih2, whh2, b2,
               wt1h, wt1m, bt1, wt2t, bt2, g_r, beta_r, out_r):
    xb = x_r[...]
    m = m_r[...]
    wihs = (wih0, wih1, wih2)
    whhs = (whh0, whh1, whh2)
    bs = (b0, b1, b2)
    seq = None
    h = None
    for layer in range(3):
        wihv = wihs[layer][...]
        whhv = whhs[layer][...]
        bv = bs[layer][...]
        h = jnp.zeros((_NB, _L), jnp.float32)
        c = jnp.zeros((_NB, _L), jnp.float32)
        nseq = []
        for t in range(_T):
            xt = xb[:, t * _F:(t + 1) * _F] if layer == 0 else seq[t]
            gates = _dot(xt, wihv) + _dot(h, whhv) + bv
            i = jax.nn.sigmoid(gates[:, :_L])
            f = jax.nn.sigmoid(gates[:, _L:2 * _L])
            gg = jnp.tanh(gates[:, 2 * _L:3 * _L])
            o = jax.nn.sigmoid(gates[:, 3 * _L:])
            c = f * c + i * gg
            h = o * jnp.tanh(c)
            nseq.append(h)
        seq = nseq
    z = _dot(h, wt1h[...]) + m * wt1m[...] + bt1[...]
    z = jnp.maximum(z, 0.0)
    z = _dot(z, wt2t[...]) + bt2[...]
    out_r[...] = _ln(z, g_r[...], beta_r[...])


def _lstm_tc(x36, mass2, params):
    ls = params["lstm"]
    tf = params["temp_fc"]
    wt1 = tf["lins"][0]["W"]
    args = []
    for lp in ls:
        args += [lp["W_ih"].T, lp["W_hh"].T, lp["b_ih"] + lp["b_hh"]]
    args += [wt1[:, :_L].T, wt1[:, _L], tf["lins"][0]["b"],
             tf["lins"][1]["W"].T, tf["lins"][1]["b"], tf["g"], tf["beta"]]
    in_specs = [pl.BlockSpec((_NB, _T * _F), lambda i: (i, 0)),
                pl.BlockSpec((_NB, 1), lambda i: (i, 0))]
    in_specs += [_wspec(a.shape) for a in args]
    return pl.pallas_call(
        _lstm_body,
        grid=(_N // _NB,),
        in_specs=in_specs,
        out_specs=pl.BlockSpec((_NB, _L), lambda i: (i, 0)),
        out_shape=jax.ShapeDtypeStruct((_N, _L), jnp.float32),
    )(x36, mass2, *args)


def _edge_enc_body(ea_r, w1t, b1, w2t, b2, g_r, beta_r, out_r):
    z = lax.dot_general(ea_r[...], w1t[...], (((0,), (0,)), ((), ())),
                        preferred_element_type=jnp.float32) + b1[...]
    z = jnp.maximum(z, 0.0)
    z = _dot(z, w2t[...]) + b2[...]
    out_r[...] = _ln(z, g_r[...], beta_r[...])


def _edge_enc_tc(edge_attr, p):
    args = [p["lins"][0]["W"].T, p["lins"][0]["b"],
            p["lins"][1]["W"].T, p["lins"][1]["b"], p["g"], p["beta"]]
    in_specs = [pl.BlockSpec((4, _EB), lambda i: (0, i))]
    in_specs += [_wspec(a.shape) for a in args]
    return pl.pallas_call(
        _edge_enc_body,
        grid=(_E // _EB,),
        in_specs=in_specs,
        out_specs=pl.BlockSpec((_EB, _L), lambda i: (i, 0)),
        out_shape=jax.ShapeDtypeStruct((_E, _L), jnp.float32),
    )(edge_attr.T, *args)


def _edge2_body(hd, hs, ef, wa, wb, wc, b1, w2t, b2, g_r, beta_r, msg_r, efo_r):
    hdv, hsv, efv = hd[...], hs[...], ef[...]
    wav, wbv, w2v = wa[...], wb[...], w2t[...]
    gv, bev = g_r[...], beta_r[...]
    pc = _dot(efv, wc[...]) + b1[...]
    z1 = jnp.maximum(_dot(hdv, wav) + _dot(hsv, wbv) + pc, 0.0)
    msg_r[...] = _ln(_dot(z1, w2v) + b2[...], gv, bev)
    z2 = jnp.maximum(_dot(hsv, wav) + _dot(hdv, wbv) + pc, 0.0)
    efo_r[...] = efv + _ln(_dot(z2, w2v) + b2[...], gv, bev)


def _edge1_body(hd, hs, ef, wa, wb, wc, b1, w2t, b2, g_r, beta_r, msg_r):
    pc = _dot(ef[...], wc[...]) + b1[...]
    z1 = jnp.maximum(_dot(hd[...], wa[...]) + _dot(hs[...], wb[...]) + pc, 0.0)
    msg_r[...] = _ln(_dot(z1, w2t[...]) + b2[...], g_r[...], beta_r[...])


def _edge_rad_body(hd, hs, wa, wb, b1, w2t, b2, g_r, beta_r, msg_r):
    z1 = jnp.maximum(_dot(hd[...], wa[...]) + _dot(hs[...], wb[...]) + b1[...],
                     0.0)
    msg_r[...] = _ln(_dot(z1, w2t[...]) + b2[...], g_r[...], beta_r[...])


def _edge_weight_args(p):
    w1 = p["lins"][0]["W"]
    return [w1[:, :_L].T, w1[:, _L:2 * _L].T, w1[:, 2 * _L:].T,
            p["lins"][0]["b"], p["lins"][1]["W"].T, p["lins"][1]["b"],
            p["g"], p["beta"]]


def _edge_mlp_tc(hd_g, hs_g, ef, p, want_ef):
    args = _edge_weight_args(p)
    in_specs = [pl.BlockSpec((_EB, _L), lambda i: (i, 0)) for _ in range(3)]
    in_specs += [_wspec(a.shape) for a in args]
    eb_spec = pl.BlockSpec((_EB, _L), lambda i: (i, 0))
    eb_shape = jax.ShapeDtypeStruct((_E, _L), jnp.float32)
    if want_ef:
        return pl.pallas_call(
            _edge2_body,
            grid=(_E // _EB,),
            in_specs=in_specs,
            out_specs=[eb_spec, eb_spec],
            out_shape=[eb_shape, eb_shape],
        )(hd_g, hs_g, ef, *args)
    return pl.pallas_call(
        _edge1_body,
        grid=(_E // _EB,),
        in_specs=in_specs,
        out_specs=eb_spec,
        out_shape=eb_shape,
    )(hd_g, hs_g, ef, *args)


def _edge_rad_tc(hd_g, hs_g, p):
    w1 = p["lins"][0]["W"]
    args = [w1[:, :_L].T, w1[:, _L:2 * _L].T, p["lins"][0]["b"],
            p["lins"][1]["W"].T, p["lins"][1]["b"], p["g"], p["beta"]]
    in_specs = [pl.BlockSpec((_EB, _L), lambda i: (i, 0)) for _ in range(2)]
    in_specs += [_wspec(a.shape) for a in args]
    return pl.pallas_call(
        _edge_rad_body,
        grid=(_E // _EB,),
        in_specs=in_specs,
        out_specs=pl.BlockSpec((_EB, _L), lambda i: (i, 0)),
        out_shape=jax.ShapeDtypeStruct((_E, _L), jnp.float32),
    )(hd_g, hs_g, *args)


def _node_body(a0, a1, h_r, wa, wb, b1, w2t, b2, g_r, beta_r, out_r):
    hv = h_r[...]
    aggr = a0[...] + a1[...]
    z = jnp.maximum(_dot(aggr, wa[...]) + _dot(hv, wb[...]) + b1[...], 0.0)
    z = _dot(z, w2t[...]) + b2[...]
    out_r[...] = hv + _ln(z, g_r[...], beta_r[...])


def _node_tc(parts, h, p):
    w1 = p["lins"][0]["W"]
    args = [w1[:, :_L].T, w1[:, _L:].T, p["lins"][0]["b"],
            p["lins"][1]["W"].T, p["lins"][1]["b"], p["g"], p["beta"]]
    nblk = _N // _NB
    in_specs = [pl.BlockSpec((_NB, _L), lambda i: (i, 0)),
                pl.BlockSpec((_NB, _L), lambda i, _o=nblk: (i + _o, 0)),
                pl.BlockSpec((_NB, _L), lambda i: (i, 0))]
    in_specs += [_wspec(a.shape) for a in args]
    return pl.pallas_call(
        _node_body,
        grid=(nblk,),
        in_specs=in_specs,
        out_specs=pl.BlockSpec((_NB, _L), lambda i: (i, 0)),
        out_shape=jax.ShapeDtypeStruct((_N, _L), jnp.float32),
    )(parts, parts, h, *args)


def _final_body(ht, hr, wapa, wapb, bap, gap, betap, wd1t, bd1, wd2t, bd2,
                gd, betad, out_r):
    z = _dot(ht[...], wapa[...]) + _dot(hr[...], wapb[...]) + bap[...]
    hcat = _ln(z, gap[...], betap[...])
    z2 = jnp.maximum(_dot(hcat, wd1t[...]) + bd1[...], 0.0)
    z2 = _dot(z2, wd2t[...]) + bd2[...]
    out_r[...] = _ln(z2, gd[...], betad[...])


def _final_tc(h_topo, h_rad, pap, pdec):
    wap = pap["lins"][0]["W"]
    args = [wap[:, :_L].T, wap[:, _L:].T, pap["lins"][0]["b"],
            pap["g"], pap["beta"],
            pdec["lins"][0]["W"].T, pdec["lins"][0]["b"],
            pdec["lins"][1]["W"].T, pdec["lins"][1]["b"],
            pdec["g"], pdec["beta"]]
    in_specs = [pl.BlockSpec((_NB, _L), lambda i: (i, 0)),
                pl.BlockSpec((_NB, _L), lambda i: (i, 0))]
    in_specs += [_wspec(a.shape) for a in args]
    return pl.pallas_call(
        _final_body,
        grid=(_N // _NB,),
        in_specs=in_specs,
        out_specs=pl.BlockSpec((_NB, _OUT), lambda i: (i, 0)),
        out_shape=jax.ShapeDtypeStruct((_N, _OUT), jnp.float32),
    )(h_topo, h_rad, *args)


def kernel(x, node_mass, edge_attr, params, edge_index, radius_edges):
    x36 = jnp.transpose(x, (0, 2, 1)).reshape(_N, _T * _F)
    mass2 = node_mass[:, None]
    src = edge_index[0].astype(jnp.int32)
    dst = edge_index[1].astype(jnp.int32)
    rsrc = radius_edges[0].astype(jnp.int32)
    rdst = radius_edges[1].astype(jnp.int32)

    h0 = _lstm_tc(x36, mass2, params)
    ef = _edge_enc_tc(edge_attr, params["edge_enc"])
    zeros_nl = jnp.zeros((_N, _L), jnp.float32)

    pe, pn = params["topo_edge"], params["topo_node"]
    h = h0

    hs_g, hd_g = _gather(h, src, dst)
    h0b, hs_g, hd_g = lax.optimization_barrier((h0, hs_g, hd_g))
    hs_r, hd_r = _gather(h0b, rsrc, rdst)
    msg, ef = _edge_mlp_tc(hd_g, hs_g, ef, pe, True)
    msg, hs_r, hd_r = lax.optimization_barrier((msg, hs_r, hd_r))
    parts = _scatter(msg, dst, zeros_nl)
    msg_r = _edge_rad_tc(hd_r, hs_r, params["rad_edge"])
    h = _node_tc(parts, h, pn)

    hs_g, hd_g = _gather(h, src, dst)
    hs_g, hd_g, msg_r = lax.optimization_barrier((hs_g, hd_g, msg_r))
    parts_r = _scatter(msg_r, rdst, zeros_nl)
    msg, ef = _edge_mlp_tc(hd_g, hs_g, ef, pe, True)
    msg, parts_r = lax.optimization_barrier((msg, parts_r))
    parts = _scatter(msg, dst, zeros_nl)
    h = _node_tc(parts, h, pn)
    h_rad = _node_tc(parts_r, h0, params["rad_node"])

    hs_g, hd_g = _gather(h, src, dst)
    msg = _edge_mlp_tc(hd_g, hs_g, ef, pe, False)
    parts = _scatter(msg, dst, zeros_nl)
    h = _node_tc(parts, h, pn)

    return _final_tc(h, h_rad, params["add_passage"], params["node_dec"])
